# baseline (speedup 1.0000x reference)
import functools

import numpy as np
import jax
import jax.numpy as jnp
from jax import lax
from jax.experimental import pallas as pl
from jax.experimental.pallas import tpu as pltpu
from jax.experimental.pallas import tpu_sc as plsc

MAX_LEN = 2048
MODEL_DIM = 768
BATCH = 4


def _build_pe(max_len, model_dim):
    pos = np.arange(max_len)[:, None]
    pe = pos / np.power(10000, 2.0 * np.arange(model_dim)[None, :] / model_dim)
    pe[:, 0::2] = np.sin(pe[:, 0::2])
    pe[:, 1::2] = np.cos(pe[:, 1::2])
    return pe.astype(np.float32)


_PE = _build_pe(MAX_LEN, MODEL_DIM)

_info = plsc.get_sparse_core_info()
_NC, _NS, _L = _info.num_cores, _info.num_subcores, _info.num_lanes
_NW = _NC * _NS
_TPW = MAX_LEN // _NW
_VPR = MODEL_DIM // _L

_mesh = plsc.VectorSubcoreMesh(core_axis_name="c", subcore_axis_name="s")


@functools.partial(
    pl.kernel,
    mesh=_mesh,
    out_type=jax.ShapeDtypeStruct((BATCH * MAX_LEN, MODEL_DIM), jnp.float32),
    scratch_types=[
        pltpu.VMEM((BATCH, _TPW), jnp.int32),
        pltpu.VMEM((_TPW, MODEL_DIM), jnp.float32),
        pltpu.VMEM((_TPW, MODEL_DIM), jnp.float32),
        pltpu.SemaphoreType.DMA,
    ],
)
def _embed(x_hbm, table_hbm, pe_hbm, out_hbm, idx_v, pe_v, rows_v, sem):
    wid = lax.axis_index("s") * _NC + lax.axis_index("c")
    t0 = wid * _TPW
    pltpu.sync_copy(pe_hbm.at[pl.ds(t0, _TPW)], pe_v)
    for b in range(BATCH):
        pltpu.sync_copy(x_hbm.at[pl.ds(b * MAX_LEN + t0, _TPW)], idx_v.at[b])
    for b in range(BATCH):
        pltpu.async_copy(table_hbm.at[idx_v.at[b]], rows_v, sem).wait()

        def row_body(r, carry):
            for j in range(_VPR):
                sl = pl.ds(j * _L, _L)
                rows_v[r, sl] = rows_v[r, sl] + pe_v[r, sl]
            return carry

        lax.fori_loop(0, _TPW, row_body, 0)
        pltpu.sync_copy(rows_v, out_hbm.at[pl.ds(b * MAX_LEN + t0, _TPW)])


def kernel(x, table):
    xf = x.reshape(-1).astype(jnp.int32)
    out = _embed(xf, table, jnp.asarray(_PE))
    return out.reshape(BATCH, MAX_LEN, MODEL_DIM)
